# baseline (speedup 1.0000x reference)
import functools

import jax
import jax.numpy as jnp
from jax.experimental import pallas as pl
from jax.experimental.pallas import tpu as pltpu

_COMMITMENT_COST = 0.25


def _row_min(a):
    c = a
    width = c.shape[1]
    while width > 128:
        half = width // 2
        c = jnp.minimum(c[:, :half], c[:, half:])
        width = half
    return jnp.min(c, axis=1, keepdims=True)


def _vq_kernel(x_ref, w_ref, enc_ref, q_ref, loss_ref, ppl_ref,
               acc_loss, acc_cnt, *, nblk, n_rows, n_codes, dim,
               block_rows, chunk_rows):
    i = pl.program_id(0)
    w = w_ref[...]
    wsq = jnp.sum(w * w, axis=1, keepdims=True).T
    col = jax.lax.broadcasted_iota(
        jnp.int32, (1, n_codes), 1).astype(jnp.float32)

    @pl.when(i == 0)
    def _init():
        acc_loss[0, 0] = 0.0
        acc_cnt[...] = jnp.zeros_like(acc_cnt)

    def _chunk(c, carry):
        r0 = c * chunk_rows
        x = x_ref[pl.ds(r0, chunk_rows), :]
        xsq = jnp.sum(x * x, axis=1, keepdims=True)
        mm2 = jax.lax.dot_general(
            2.0 * x, w, (((1,), (1,)), ((), ())),
            preferred_element_type=jnp.float32)
        d = (xsq + wsq) - mm2
        dmin = _row_min(d)
        idx = _row_min(jnp.where(d == dmin, col, float(n_codes)))
        enc = (col == idx).astype(jnp.float32)
        enc_ref[pl.ds(r0, chunk_rows), :] = enc
        q = jax.lax.dot_general(
            enc, w, (((1,), (0,)), ((), ())),
            preferred_element_type=jnp.float32)
        diff = q - x
        q_ref[pl.ds(r0, chunk_rows), :] = x + diff
        acc_loss[0, 0] += jnp.sum(diff * diff)
        ones_row = jnp.ones((1, chunk_rows), jnp.float32)
        acc_cnt[...] += jax.lax.dot_general(
            ones_row, enc, (((1,), (0,)), ((), ())),
            preferred_element_type=jnp.float32)
        return carry

    jax.lax.fori_loop(0, block_rows // chunk_rows, _chunk, 0)

    @pl.when(i == nblk - 1)
    def _fini():
        mean_sq = acc_loss[0, 0] / (n_rows * dim)
        loss_ref[...] = jnp.full((1, 1), mean_sq + _COMMITMENT_COST * mean_sq,
                                 jnp.float32)
        p = acc_cnt[...] / n_rows
        ent = jnp.sum(p * jnp.log(p + 1e-10), axis=1, keepdims=True)
        ppl_ref[...] = jnp.exp(-ent)


def kernel(inputs, weight):
    n, dim = inputs.shape
    n_codes = weight.shape[0]
    block_rows = 2048 if n % 2048 == 0 else n
    chunk_rows = 256 if block_rows % 256 == 0 else block_rows
    nblk = n // block_rows

    enc, q_ste, loss, ppl = pl.pallas_call(
        functools.partial(_vq_kernel, nblk=nblk, n_rows=n,
                          n_codes=n_codes, dim=dim,
                          block_rows=block_rows, chunk_rows=chunk_rows),
        grid=(nblk,),
        in_specs=[
            pl.BlockSpec((block_rows, dim), lambda i: (i, 0)),
            pl.BlockSpec((n_codes, dim), lambda i: (0, 0)),
        ],
        out_specs=[
            pl.BlockSpec((block_rows, n_codes), lambda i: (i, 0)),
            pl.BlockSpec((block_rows, dim), lambda i: (i, 0)),
            pl.BlockSpec((1, 1), lambda i: (0, 0)),
            pl.BlockSpec((1, 1), lambda i: (0, 0)),
        ],
        out_shape=[
            jax.ShapeDtypeStruct((n, n_codes), jnp.float32),
            jax.ShapeDtypeStruct((n, dim), jnp.float32),
            jax.ShapeDtypeStruct((1, 1), jnp.float32),
            jax.ShapeDtypeStruct((1, 1), jnp.float32),
        ],
        scratch_shapes=[
            pltpu.SMEM((1, 1), jnp.float32),
            pltpu.VMEM((1, n_codes), jnp.float32),
        ],
    )(inputs, weight)

    return (loss.reshape(()), q_ste, ppl.reshape(()), enc)
